# baseline (speedup 1.0000x reference)
import jax
import jax.numpy as jnp
from jax.experimental import pallas as pl
from jax.experimental.pallas import tpu as pltpu

_BT = 128


def _conv1_lhs(x):
    n = x.shape[0]
    xp = jnp.pad(x.reshape(n, 28, 28), ((0, 0), (1, 7), (1, 1)))
    pieces = []
    for j in range(4):
        for kh in range(3):
            r = j + kh
            pieces.append(jax.lax.slice(xp, (0, r, 0), (n, r + 29, 30), (1, 4, 1)))
    st = jnp.stack(pieces, axis=2)
    out = st.reshape(n, 8, 4, 90).transpose(0, 2, 1, 3).reshape(n, 32, 90)
    return out.astype(jnp.bfloat16)


def _toeplitz1(w1f):
    xe = jnp.arange(0, 28, 2)
    xo = jnp.arange(1, 28, 2)
    xin = jnp.arange(30)

    def block(xs):
        d = xin[:, None] - xs[None, :]
        valid = (d >= 0) & (d <= 2)
        kh = jnp.arange(3)[:, None, None]
        tap = kh * 3 + jnp.clip(d, 0, 2)[None]
        t = w1f[tap, 0, :] * valid[None, :, :, None]
        z = jnp.zeros((3, 30, 1, 32), w1f.dtype)
        return jnp.concatenate([z, t, z], axis=2)

    return jnp.concatenate([block(xe), block(xo)], axis=2).reshape(90, 1024)


def _toeplitz2(w2f):
    xe = jnp.arange(0, 14, 2)
    xo = jnp.arange(1, 14, 2)
    xin = jnp.arange(16)

    def block(kh, xs):
        d = xin[:, None] - xs[None, :]
        valid = (d >= 0) & (d <= 2)
        tap = kh * 3 + jnp.clip(d, 0, 2)
        t = w2f[tap] * valid[:, :, None, None]
        t = t.transpose(0, 2, 1, 3)
        z = jnp.zeros((16, 32, 1, 64), w2f.dtype)
        return jnp.concatenate([t, z], axis=2)

    return jnp.concatenate([
        jnp.concatenate([block(kh, xe), block(kh, xo)], axis=2).reshape(512, 1024)
        for kh in range(3)
    ], axis=0)


def _fused_cnn_kernel(l1_ref, b1_ref, s1_ref, b2_ref,
                      s2_ref, fw1_ref, fb1_ref, fw2_ref, fb2_ref, o_ref):
    bsz = l1_ref.shape[0]
    bf16 = jnp.bfloat16
    lhs1 = l1_ref[...].reshape(bsz * 32, 90)
    c1 = jnp.dot(lhs1, b1_ref[...],
                 preferred_element_type=jnp.float32).astype(bf16)
    c1 = c1.reshape(bsz, 32, 1024)
    mx = jnp.maximum(c1[..., :512], c1[..., 512:])
    zero = jnp.zeros((), bf16)
    pe = jnp.maximum(jnp.maximum(mx[:, 0:8], mx[:, 8:16]) + s1_ref[...], zero)
    po = jnp.maximum(jnp.maximum(mx[:, 16:24], mx[:, 24:32]) + s1_ref[...], zero)

    z1 = jnp.zeros((bsz, 1, 512), bf16)
    re = jnp.concatenate([z1, po[:, 0:7]], axis=1)
    ro = jnp.concatenate([pe[:, 0:7], z1], axis=1)
    re1 = jnp.concatenate([po[:, 0:7], z1], axis=1)
    ro1 = jnp.concatenate([pe[:, 1:7], z1, z1], axis=1)

    f32 = jnp.float32
    l2e = jnp.concatenate([re, ro, re1], axis=-1)
    l2o = jnp.concatenate([ro, re1, ro1], axis=-1)
    lhs2 = jnp.concatenate([l2e, l2o], axis=1).reshape(bsz * 16, 1536)
    c2 = jnp.dot(lhs2, b2_ref[...],
                 preferred_element_type=f32).astype(bf16)
    c2 = c2.reshape(bsz, 16, 1024)
    m2 = jnp.maximum(c2[:, 0:8], c2[:, 8:16])
    m2 = jnp.maximum(m2[..., :512], m2[..., 512:])
    p2 = jnp.maximum(m2 + s2_ref[...], zero)

    flat = p2.reshape(bsz, 4096)
    h1 = jnp.dot(flat, fw1_ref[...],
                 preferred_element_type=f32) + fb1_ref[...]
    h1 = jnp.maximum(h1, 0.0).astype(bf16)
    out = jnp.dot(h1, fw2_ref[...],
                  preferred_element_type=f32) + fb2_ref[...]
    o_ref[...] = out


def kernel(x, w1f, shift1, w2f, shift2, fc1_w, fc1_b, fc2_w, fc2_b):
    n = x.shape[0]
    lhs1 = _conv1_lhs(x)
    np_ = ((n + _BT - 1) // _BT) * _BT
    if np_ != n:
        lhs1 = jnp.pad(lhs1, ((0, np_ - n), (0, 0), (0, 0)))
    bf16 = jnp.bfloat16
    b1 = _toeplitz1(w1f).astype(bf16)
    b2 = _toeplitz2(w2f).astype(bf16)
    zc = jnp.zeros((1, 32), shift1.dtype)
    s1 = jnp.concatenate([zc, jnp.tile(shift1, (1, 14)), zc],
                         axis=1).reshape(1, 1, 512).astype(bf16)
    zc2 = jnp.zeros((1, 64), shift2.dtype)
    s2 = jnp.concatenate([jnp.tile(shift2, (1, 7)), zc2],
                         axis=1).reshape(1, 1, 512).astype(bf16)
    fw1 = jnp.concatenate([fc1_w, jnp.zeros((512, fc1_w.shape[1]), fc1_w.dtype)],
                          axis=0).astype(bf16)
    fw2 = fc2_w.astype(bf16)
    out = pl.pallas_call(
        _fused_cnn_kernel,
        out_shape=jax.ShapeDtypeStruct((np_, 10), jnp.float32),
        grid=(np_ // _BT,),
        in_specs=[
            pl.BlockSpec((_BT, 32, 90), lambda i: (i, 0, 0)),
            pl.BlockSpec((90, 1024), lambda i: (0, 0)),
            pl.BlockSpec((1, 1, 512), lambda i: (0, 0, 0)),
            pl.BlockSpec((1536, 1024), lambda i: (0, 0)),
            pl.BlockSpec((1, 1, 512), lambda i: (0, 0, 0)),
            pl.BlockSpec((4096, 128), lambda i: (0, 0)),
            pl.BlockSpec((1, 128), lambda i: (0, 0)),
            pl.BlockSpec((128, 10), lambda i: (0, 0)),
            pl.BlockSpec((1, 10), lambda i: (0, 0)),
        ],
        out_specs=pl.BlockSpec((_BT, 10), lambda i: (i, 0)),
        compiler_params=pltpu.CompilerParams(
            dimension_semantics=("arbitrary",),
            vmem_limit_bytes=56 * 1024 * 1024),
    )(lhs1, b1, s1, b2, s2, fw1, fc1_b, fw2, fc2_b)
    return out[:n] if np_ != n else out


# speedup vs baseline: 1.0017x; 1.0017x over previous
import jax
import jax.numpy as jnp
from jax.experimental import pallas as pl
from jax.experimental.pallas import tpu as pltpu

_BT = 64


def _conv1_lhs(x):
    n = x.shape[0]
    xp = jnp.pad(x.reshape(n, 28, 28), ((0, 0), (1, 7), (1, 1)))
    pieces = []
    for j in range(4):
        for kh in range(3):
            r = j + kh
            pieces.append(jax.lax.slice(xp, (0, r, 0), (n, r + 29, 30), (1, 4, 1)))
    st = jnp.stack(pieces, axis=2)
    out = st.reshape(n, 8, 4, 90).transpose(0, 2, 1, 3).reshape(n, 32, 90)
    return out.astype(jnp.bfloat16)


def _toeplitz1(w1f):
    xe = jnp.arange(0, 28, 2)
    xo = jnp.arange(1, 28, 2)
    xin = jnp.arange(30)

    def block(xs):
        d = xin[:, None] - xs[None, :]
        valid = (d >= 0) & (d <= 2)
        kh = jnp.arange(3)[:, None, None]
        tap = kh * 3 + jnp.clip(d, 0, 2)[None]
        t = w1f[tap, 0, :] * valid[None, :, :, None]
        z = jnp.zeros((3, 30, 1, 32), w1f.dtype)
        return jnp.concatenate([z, t, z], axis=2)

    return jnp.concatenate([block(xe), block(xo)], axis=2).reshape(90, 1024)


def _toeplitz2(w2f):
    xe = jnp.arange(0, 14, 2)
    xo = jnp.arange(1, 14, 2)
    xin = jnp.arange(16)

    def block(kh, xs):
        d = xin[:, None] - xs[None, :]
        valid = (d >= 0) & (d <= 2)
        tap = kh * 3 + jnp.clip(d, 0, 2)
        t = w2f[tap] * valid[:, :, None, None]
        t = t.transpose(0, 2, 1, 3)
        z = jnp.zeros((16, 32, 1, 64), w2f.dtype)
        return jnp.concatenate([t, z], axis=2)

    return jnp.concatenate([
        jnp.concatenate([block(kh, xe), block(kh, xo)], axis=2).reshape(512, 1024)
        for kh in range(3)
    ], axis=0)


def _fused_cnn_kernel(l1_ref, b1_ref, s1_ref, b2_ref,
                      s2_ref, fw1_ref, fb1_ref, fw2_ref, fb2_ref, o_ref):
    bsz = l1_ref.shape[0]
    bf16 = jnp.bfloat16
    lhs1 = l1_ref[...].reshape(bsz * 32, 90)
    c1 = jnp.dot(lhs1, b1_ref[...],
                 preferred_element_type=jnp.float32).astype(bf16)
    c1 = c1.reshape(bsz, 32, 1024)
    mx = jnp.maximum(c1[..., :512], c1[..., 512:])
    zero = jnp.zeros((), bf16)
    pe = jnp.maximum(jnp.maximum(mx[:, 0:8], mx[:, 8:16]) + s1_ref[...], zero)
    po = jnp.maximum(jnp.maximum(mx[:, 16:24], mx[:, 24:32]) + s1_ref[...], zero)

    z1 = jnp.zeros((bsz, 1, 512), bf16)
    re = jnp.concatenate([z1, po[:, 0:7]], axis=1)
    ro = jnp.concatenate([pe[:, 0:7], z1], axis=1)
    re1 = jnp.concatenate([po[:, 0:7], z1], axis=1)
    ro1 = jnp.concatenate([pe[:, 1:7], z1, z1], axis=1)

    f32 = jnp.float32
    l2e = jnp.concatenate([re, ro, re1], axis=-1)
    l2o = jnp.concatenate([ro, re1, ro1], axis=-1)
    lhs2 = jnp.concatenate([l2e, l2o], axis=1).reshape(bsz * 16, 1536)
    c2 = jnp.dot(lhs2, b2_ref[...],
                 preferred_element_type=f32).astype(bf16)
    c2 = c2.reshape(bsz, 16, 1024)
    m2 = jnp.maximum(c2[:, 0:8], c2[:, 8:16])
    m2 = jnp.maximum(m2[..., :512], m2[..., 512:])
    p2 = jnp.maximum(m2 + s2_ref[...], zero)

    flat = p2.reshape(bsz, 4096)
    h1 = jnp.dot(flat, fw1_ref[...],
                 preferred_element_type=f32) + fb1_ref[...]
    h1 = jnp.maximum(h1, 0.0).astype(bf16)
    out = jnp.dot(h1, fw2_ref[...],
                  preferred_element_type=f32) + fb2_ref[...]
    o_ref[...] = out


def kernel(x, w1f, shift1, w2f, shift2, fc1_w, fc1_b, fc2_w, fc2_b):
    n = x.shape[0]
    lhs1 = _conv1_lhs(x)
    np_ = ((n + _BT - 1) // _BT) * _BT
    if np_ != n:
        lhs1 = jnp.pad(lhs1, ((0, np_ - n), (0, 0), (0, 0)))
    bf16 = jnp.bfloat16
    b1 = _toeplitz1(w1f).astype(bf16)
    b2 = _toeplitz2(w2f).astype(bf16)
    zc = jnp.zeros((1, 32), shift1.dtype)
    s1 = jnp.concatenate([zc, jnp.tile(shift1, (1, 14)), zc],
                         axis=1).reshape(1, 1, 512).astype(bf16)
    zc2 = jnp.zeros((1, 64), shift2.dtype)
    s2 = jnp.concatenate([jnp.tile(shift2, (1, 7)), zc2],
                         axis=1).reshape(1, 1, 512).astype(bf16)
    fw1 = jnp.concatenate([fc1_w, jnp.zeros((512, fc1_w.shape[1]), fc1_w.dtype)],
                          axis=0).astype(bf16)
    fw2 = fc2_w.astype(bf16)
    out = pl.pallas_call(
        _fused_cnn_kernel,
        out_shape=jax.ShapeDtypeStruct((np_, 10), jnp.float32),
        grid=(np_ // _BT,),
        in_specs=[
            pl.BlockSpec((_BT, 32, 90), lambda i: (i, 0, 0)),
            pl.BlockSpec((90, 1024), lambda i: (0, 0)),
            pl.BlockSpec((1, 1, 512), lambda i: (0, 0, 0)),
            pl.BlockSpec((1536, 1024), lambda i: (0, 0)),
            pl.BlockSpec((1, 1, 512), lambda i: (0, 0, 0)),
            pl.BlockSpec((4096, 128), lambda i: (0, 0)),
            pl.BlockSpec((1, 128), lambda i: (0, 0)),
            pl.BlockSpec((128, 10), lambda i: (0, 0)),
            pl.BlockSpec((1, 10), lambda i: (0, 0)),
        ],
        out_specs=pl.BlockSpec((_BT, 10), lambda i: (i, 0)),
        compiler_params=pltpu.CompilerParams(
            dimension_semantics=("arbitrary",),
            vmem_limit_bytes=56 * 1024 * 1024),
    )(lhs1, b1, s1, b2, s2, fw1, fc1_b, fw2, fc2_b)
    return out[:n] if np_ != n else out


# speedup vs baseline: 1.1665x; 1.1645x over previous
import jax
import jax.numpy as jnp
from jax.experimental import pallas as pl
from jax.experimental.pallas import tpu as pltpu

_BT = 64


def _pad_input(x):
    n = x.shape[0]
    return jnp.pad(x.astype(jnp.bfloat16).reshape(n, 28, 28),
                   ((0, 0), (1, 7), (1, 3)))


def _toeplitz1(w1f):
    xe = jnp.arange(0, 28, 2)
    xo = jnp.arange(1, 28, 2)
    xin = jnp.arange(32)

    def block(xs):
        d = xin[:, None] - xs[None, :]
        valid = (d >= 0) & (d <= 2)
        kh = jnp.arange(3)[:, None, None]
        tap = kh * 3 + jnp.clip(d, 0, 2)[None]
        t = w1f[tap, 0, :] * valid[None, :, :, None]
        z = jnp.zeros((3, 32, 1, 32), w1f.dtype)
        return jnp.concatenate([z, t, z], axis=2)

    return jnp.concatenate([block(xe), block(xo)], axis=2).reshape(96, 1024)


def _toeplitz2(w2f):
    xe = jnp.arange(0, 14, 2)
    xo = jnp.arange(1, 14, 2)
    xin = jnp.arange(16)

    def block(kh, xs):
        d = xin[:, None] - xs[None, :]
        valid = (d >= 0) & (d <= 2)
        tap = kh * 3 + jnp.clip(d, 0, 2)
        t = w2f[tap] * valid[:, :, None, None]
        t = t.transpose(0, 2, 1, 3)
        z = jnp.zeros((16, 32, 1, 64), w2f.dtype)
        return jnp.concatenate([t, z], axis=2)

    return jnp.concatenate([
        jnp.concatenate([block(kh, xe), block(kh, xo)], axis=2).reshape(512, 1024)
        for kh in range(3)
    ], axis=0)


def _fused_cnn_kernel(l1_ref, b1_ref, s1_ref, b2_ref,
                      s2_ref, fw1_ref, fb1_ref, fw2_ref, fb2_ref, o_ref):
    bsz = l1_ref.shape[0]
    bf16 = jnp.bfloat16
    xr = l1_ref[...].reshape(bsz, 9, 4, 32)
    pc = [xr[:, (q // 4):(q // 4 + 8), q % 4, :] for q in range(6)]
    banks = [jnp.concatenate([pc[j], pc[j + 1], pc[j + 2]], axis=-1)
             for j in range(4)]
    lhs1 = jnp.concatenate(banks, axis=1).reshape(bsz * 32, 96)
    c1 = jnp.dot(lhs1, b1_ref[...],
                 preferred_element_type=jnp.float32).astype(bf16)
    c1 = c1.reshape(bsz, 32, 1024)
    mx = jnp.maximum(c1[..., :512], c1[..., 512:])
    zero = jnp.zeros((), bf16)
    pe = jnp.maximum(jnp.maximum(mx[:, 0:8], mx[:, 8:16]) + s1_ref[...], zero)
    po = jnp.maximum(jnp.maximum(mx[:, 16:24], mx[:, 24:32]) + s1_ref[...], zero)

    z1 = jnp.zeros((bsz, 1, 512), bf16)
    re = jnp.concatenate([z1, po[:, 0:7]], axis=1)
    ro = jnp.concatenate([pe[:, 0:7], z1], axis=1)
    re1 = jnp.concatenate([po[:, 0:7], z1], axis=1)
    ro1 = jnp.concatenate([pe[:, 1:7], z1, z1], axis=1)

    f32 = jnp.float32
    l2e = jnp.concatenate([re, ro, re1], axis=-1)
    l2o = jnp.concatenate([ro, re1, ro1], axis=-1)
    lhs2 = jnp.concatenate([l2e, l2o], axis=1).reshape(bsz * 16, 1536)
    c2 = jnp.dot(lhs2, b2_ref[...],
                 preferred_element_type=f32).astype(bf16)
    c2 = c2.reshape(bsz, 16, 1024)
    m2 = jnp.maximum(c2[:, 0:8], c2[:, 8:16])
    m2 = jnp.maximum(m2[..., :512], m2[..., 512:])
    p2 = jnp.maximum(m2 + s2_ref[...], zero)

    flat = p2.reshape(bsz, 4096)
    h1 = jnp.dot(flat, fw1_ref[...],
                 preferred_element_type=f32) + fb1_ref[...]
    h1 = jnp.maximum(h1, 0.0).astype(bf16)
    out = jnp.dot(h1, fw2_ref[...],
                  preferred_element_type=f32) + fb2_ref[...]
    o_ref[...] = out


def kernel(x, w1f, shift1, w2f, shift2, fc1_w, fc1_b, fc2_w, fc2_b):
    n = x.shape[0]
    xpad = _pad_input(x)
    np_ = ((n + _BT - 1) // _BT) * _BT
    if np_ != n:
        xpad = jnp.pad(xpad, ((0, np_ - n), (0, 0), (0, 0)))
    bf16 = jnp.bfloat16
    b1 = _toeplitz1(w1f).astype(bf16)
    b2 = _toeplitz2(w2f).astype(bf16)
    zc = jnp.zeros((1, 32), shift1.dtype)
    s1 = jnp.concatenate([zc, jnp.tile(shift1, (1, 14)), zc],
                         axis=1).reshape(1, 1, 512).astype(bf16)
    zc2 = jnp.zeros((1, 64), shift2.dtype)
    s2 = jnp.concatenate([jnp.tile(shift2, (1, 7)), zc2],
                         axis=1).reshape(1, 1, 512).astype(bf16)
    fw1 = jnp.concatenate([fc1_w, jnp.zeros((512, fc1_w.shape[1]), fc1_w.dtype)],
                          axis=0).astype(bf16)
    fw2 = fc2_w.astype(bf16)
    out = pl.pallas_call(
        _fused_cnn_kernel,
        out_shape=jax.ShapeDtypeStruct((np_, 10), jnp.float32),
        grid=(np_ // _BT,),
        in_specs=[
            pl.BlockSpec((_BT, 36, 32), lambda i: (i, 0, 0)),
            pl.BlockSpec((96, 1024), lambda i: (0, 0)),
            pl.BlockSpec((1, 1, 512), lambda i: (0, 0, 0)),
            pl.BlockSpec((1536, 1024), lambda i: (0, 0)),
            pl.BlockSpec((1, 1, 512), lambda i: (0, 0, 0)),
            pl.BlockSpec((4096, 128), lambda i: (0, 0)),
            pl.BlockSpec((1, 128), lambda i: (0, 0)),
            pl.BlockSpec((128, 10), lambda i: (0, 0)),
            pl.BlockSpec((1, 10), lambda i: (0, 0)),
        ],
        out_specs=pl.BlockSpec((_BT, 10), lambda i: (i, 0)),
        compiler_params=pltpu.CompilerParams(
            dimension_semantics=("arbitrary",),
            vmem_limit_bytes=56 * 1024 * 1024),
    )(xpad, b1, s1, b2, s2, fw1, fc1_b, fw2, fc2_b)
    return out[:n] if np_ != n else out


# speedup vs baseline: 1.2544x; 1.0754x over previous
import jax
import jax.numpy as jnp
from jax.experimental import pallas as pl
from jax.experimental.pallas import tpu as pltpu

_BT = 64


def _toeplitz1(w1f):
    xe = jnp.arange(0, 28, 2)
    xo = jnp.arange(1, 28, 2)
    xin = jnp.arange(32)

    def block(xs):
        d = xin[:, None] - xs[None, :]
        valid = (d >= 0) & (d <= 2)
        kh = jnp.arange(3)[:, None, None]
        tap = kh * 3 + jnp.clip(d, 0, 2)[None]
        t = w1f[tap, 0, :] * valid[None, :, :, None]
        z = jnp.zeros((3, 32, 1, 32), w1f.dtype)
        return jnp.concatenate([z, t, z], axis=2)

    return jnp.concatenate([block(xe), block(xo)], axis=2).reshape(96, 1024)


def _toeplitz2(w2f):
    xe = jnp.arange(0, 14, 2)
    xo = jnp.arange(1, 14, 2)
    xin = jnp.arange(16)

    def block(kh, xs):
        d = xin[:, None] - xs[None, :]
        valid = (d >= 0) & (d <= 2)
        tap = kh * 3 + jnp.clip(d, 0, 2)
        t = w2f[tap] * valid[:, :, None, None]
        t = t.transpose(0, 2, 1, 3)
        z = jnp.zeros((16, 32, 1, 64), w2f.dtype)
        return jnp.concatenate([t, z], axis=2)

    return jnp.concatenate([
        jnp.concatenate([block(kh, xe), block(kh, xo)], axis=2).reshape(512, 1024)
        for kh in range(3)
    ], axis=0)


def _fused_cnn_kernel(l1_ref, b1_ref, s1_ref, b2_ref,
                      s2_ref, fw1_ref, fb1_ref, fw2_ref, fb2_ref, o_ref):
    bsz = l1_ref.shape[0]
    bf16 = jnp.bfloat16
    xv = l1_ref[...].astype(bf16)
    zl = jnp.zeros((bsz, 28, 1), bf16)
    zr = jnp.zeros((bsz, 28, 3), bf16)
    xv = jnp.concatenate([zl, xv, zr], axis=-1)
    zt = jnp.zeros((bsz, 1, 32), bf16)
    zb = jnp.zeros((bsz, 7, 32), bf16)
    xp = jnp.concatenate([zt, xv, zb], axis=1)
    xr = xp.reshape(bsz, 9, 4, 32)
    pc = [xr[:, (q // 4):(q // 4 + 8), q % 4, :] for q in range(6)]
    banks = [jnp.concatenate([pc[j], pc[j + 1], pc[j + 2]], axis=-1)
             for j in range(4)]
    lhs1 = jnp.concatenate(banks, axis=1).reshape(bsz * 32, 96)
    c1 = jnp.dot(lhs1, b1_ref[...],
                 preferred_element_type=jnp.float32).astype(bf16)
    c1 = c1.reshape(bsz, 32, 1024)
    mx = jnp.maximum(c1[..., :512], c1[..., 512:])
    zero = jnp.zeros((), bf16)
    pe = jnp.maximum(jnp.maximum(mx[:, 0:8], mx[:, 8:16]) + s1_ref[...], zero)
    po = jnp.maximum(jnp.maximum(mx[:, 16:24], mx[:, 24:32]) + s1_ref[...], zero)

    z1 = jnp.zeros((bsz, 1, 512), bf16)
    re = jnp.concatenate([z1, po[:, 0:7]], axis=1)
    ro = jnp.concatenate([pe[:, 0:7], z1], axis=1)
    re1 = jnp.concatenate([po[:, 0:7], z1], axis=1)
    ro1 = jnp.concatenate([pe[:, 1:7], z1, z1], axis=1)

    f32 = jnp.float32
    l2e = jnp.concatenate([re, ro, re1], axis=-1)
    l2o = jnp.concatenate([ro, re1, ro1], axis=-1)
    lhs2 = jnp.concatenate([l2e, l2o], axis=1).reshape(bsz * 16, 1536)
    c2 = jnp.dot(lhs2, b2_ref[...],
                 preferred_element_type=f32).astype(bf16)
    c2 = c2.reshape(bsz, 16, 1024)
    m2 = jnp.maximum(c2[:, 0:8], c2[:, 8:16])
    m2 = jnp.maximum(m2[..., :512], m2[..., 512:])
    p2 = jnp.maximum(m2 + s2_ref[...], zero)

    flat = p2.reshape(bsz, 4096)
    h1 = jnp.dot(flat, fw1_ref[...],
                 preferred_element_type=f32) + fb1_ref[...]
    h1 = jnp.maximum(h1, 0.0).astype(bf16)
    out = jnp.dot(h1, fw2_ref[...],
                  preferred_element_type=f32) + fb2_ref[...]
    o_ref[...] = out


def kernel(x, w1f, shift1, w2f, shift2, fc1_w, fc1_b, fc2_w, fc2_b):
    n = x.shape[0]
    xpad = x.reshape(n, 28, 28)
    np_ = ((n + _BT - 1) // _BT) * _BT
    if np_ != n:
        xpad = jnp.pad(xpad, ((0, np_ - n), (0, 0), (0, 0)))
    bf16 = jnp.bfloat16
    b1 = _toeplitz1(w1f).astype(bf16)
    b2 = _toeplitz2(w2f).astype(bf16)
    zc = jnp.zeros((1, 32), shift1.dtype)
    s1 = jnp.concatenate([zc, jnp.tile(shift1, (1, 14)), zc],
                         axis=1).reshape(1, 1, 512).astype(bf16)
    zc2 = jnp.zeros((1, 64), shift2.dtype)
    s2 = jnp.concatenate([jnp.tile(shift2, (1, 7)), zc2],
                         axis=1).reshape(1, 1, 512).astype(bf16)
    fw1 = jnp.concatenate([fc1_w, jnp.zeros((512, fc1_w.shape[1]), fc1_w.dtype)],
                          axis=0).astype(bf16)
    fw2 = fc2_w.astype(bf16)
    out = pl.pallas_call(
        _fused_cnn_kernel,
        out_shape=jax.ShapeDtypeStruct((np_, 10), jnp.float32),
        grid=(np_ // _BT,),
        in_specs=[
            pl.BlockSpec((_BT, 28, 28), lambda i: (i, 0, 0)),
            pl.BlockSpec((96, 1024), lambda i: (0, 0)),
            pl.BlockSpec((1, 1, 512), lambda i: (0, 0, 0)),
            pl.BlockSpec((1536, 1024), lambda i: (0, 0)),
            pl.BlockSpec((1, 1, 512), lambda i: (0, 0, 0)),
            pl.BlockSpec((4096, 128), lambda i: (0, 0)),
            pl.BlockSpec((1, 128), lambda i: (0, 0)),
            pl.BlockSpec((128, 10), lambda i: (0, 0)),
            pl.BlockSpec((1, 10), lambda i: (0, 0)),
        ],
        out_specs=pl.BlockSpec((_BT, 10), lambda i: (i, 0)),
        compiler_params=pltpu.CompilerParams(
            dimension_semantics=("arbitrary",),
            vmem_limit_bytes=56 * 1024 * 1024),
    )(xpad, b1, s1, b2, s2, fw1, fc1_b, fw2, fc2_b)
    return out[:n] if np_ != n else out


# speedup vs baseline: 1.3943x; 1.1115x over previous
import jax
import jax.numpy as jnp
from jax.experimental import pallas as pl
from jax.experimental.pallas import tpu as pltpu

_BT = 64


def _toeplitz1(w1f):
    xe = jnp.arange(0, 28, 2)
    xo = jnp.arange(1, 28, 2)
    xin = jnp.arange(32)

    def block(xs):
        d = xin[:, None] - xs[None, :]
        valid = (d >= 0) & (d <= 2)
        kh = jnp.arange(3)[:, None, None]
        tap = kh * 3 + jnp.clip(d, 0, 2)[None]
        t = w1f[tap, 0, :] * valid[None, :, :, None]
        z = jnp.zeros((3, 32, 1, 32), w1f.dtype)
        return jnp.concatenate([z, t, z], axis=2)

    return jnp.concatenate([block(xe), block(xo)], axis=2).reshape(96, 1024)


def _toeplitz2(w2f):
    xe = jnp.arange(0, 14, 2)
    xo = jnp.arange(1, 14, 2)
    xin = jnp.arange(16)

    def block(kh, xs):
        d = xin[:, None] - xs[None, :]
        valid = (d >= 0) & (d <= 2)
        tap = kh * 3 + jnp.clip(d, 0, 2)
        t = w2f[tap] * valid[:, :, None, None]
        t = t.transpose(0, 2, 1, 3)
        z = jnp.zeros((16, 32, 1, 64), w2f.dtype)
        return jnp.concatenate([t, z], axis=2)

    return jnp.concatenate([
        jnp.concatenate([block(kh, xe), block(kh, xo)], axis=2).reshape(512, 1024)
        for kh in range(3)
    ], axis=0)


def _fused_cnn_kernel(l1_ref, b1_ref, s1_ref, b2_ref,
                      s2_ref, fw1_ref, fb1_ref, fw2_ref, fb2_ref, o_ref):
    bsz = l1_ref.shape[0]
    bf16 = jnp.bfloat16
    xv = l1_ref[...].astype(bf16)
    zl = jnp.zeros((bsz, 28, 1), bf16)
    zr = jnp.zeros((bsz, 28, 3), bf16)
    xv = jnp.concatenate([zl, xv, zr], axis=-1)
    zt = jnp.zeros((bsz, 1, 32), bf16)
    zb = jnp.zeros((bsz, 7, 32), bf16)
    xp = jnp.concatenate([zt, xv, zb], axis=1)
    xr = xp.reshape(bsz, 9, 4, 32).transpose(0, 2, 1, 3)
    pc = [xr[:, q % 4, (q // 4):(q // 4 + 8), :] for q in range(6)]
    banks = [jnp.concatenate([pc[j], pc[j + 1], pc[j + 2]], axis=-1)
             for j in range(4)]
    lhs1 = jnp.concatenate(banks, axis=1).reshape(bsz * 32, 96)
    c1 = jnp.dot(lhs1, b1_ref[...],
                 preferred_element_type=jnp.float32).astype(bf16)
    c1 = c1.reshape(bsz, 32, 1024)
    mx = jnp.maximum(c1[..., :512], c1[..., 512:])
    zero = jnp.zeros((), bf16)
    pe = jnp.maximum(jnp.maximum(mx[:, 0:8], mx[:, 8:16]) + s1_ref[...], zero)
    po = jnp.maximum(jnp.maximum(mx[:, 16:24], mx[:, 24:32]) + s1_ref[...], zero)

    z1 = jnp.zeros((bsz, 1, 512), bf16)
    re = jnp.concatenate([z1, po[:, 0:7]], axis=1)
    ro = jnp.concatenate([pe[:, 0:7], z1], axis=1)
    re1 = jnp.concatenate([po[:, 0:7], z1], axis=1)
    ro1 = jnp.concatenate([pe[:, 1:7], z1, z1], axis=1)

    f32 = jnp.float32
    l2e = jnp.concatenate([re, ro, re1], axis=-1)
    l2o = jnp.concatenate([ro, re1, ro1], axis=-1)
    lhs2 = jnp.concatenate([l2e, l2o], axis=1).reshape(bsz * 16, 1536)
    c2 = jnp.dot(lhs2, b2_ref[...],
                 preferred_element_type=f32).astype(bf16)
    c2 = c2.reshape(bsz, 16, 1024)
    m2 = jnp.maximum(c2[:, 0:8], c2[:, 8:16])
    m2 = jnp.maximum(m2[..., :512], m2[..., 512:])
    p2 = jnp.maximum(m2 + s2_ref[...], zero)

    flat = p2.reshape(bsz, 4096)
    h1 = jnp.dot(flat, fw1_ref[...],
                 preferred_element_type=f32) + fb1_ref[...]
    h1 = jnp.maximum(h1, 0.0).astype(bf16)
    out = jnp.dot(h1, fw2_ref[...],
                  preferred_element_type=f32) + fb2_ref[...]
    o_ref[...] = out


def kernel(x, w1f, shift1, w2f, shift2, fc1_w, fc1_b, fc2_w, fc2_b):
    n = x.shape[0]
    xpad = x.reshape(n, 28, 28)
    np_ = ((n + _BT - 1) // _BT) * _BT
    if np_ != n:
        xpad = jnp.pad(xpad, ((0, np_ - n), (0, 0), (0, 0)))
    bf16 = jnp.bfloat16
    b1 = _toeplitz1(w1f).astype(bf16)
    b2 = _toeplitz2(w2f).astype(bf16)
    zc = jnp.zeros((1, 32), shift1.dtype)
    s1 = jnp.concatenate([zc, jnp.tile(shift1, (1, 14)), zc],
                         axis=1).reshape(1, 1, 512).astype(bf16)
    zc2 = jnp.zeros((1, 64), shift2.dtype)
    s2 = jnp.concatenate([jnp.tile(shift2, (1, 7)), zc2],
                         axis=1).reshape(1, 1, 512).astype(bf16)
    fw1 = jnp.concatenate([fc1_w, jnp.zeros((512, fc1_w.shape[1]), fc1_w.dtype)],
                          axis=0).astype(bf16)
    fw2 = fc2_w.astype(bf16)
    out = pl.pallas_call(
        _fused_cnn_kernel,
        out_shape=jax.ShapeDtypeStruct((np_, 10), jnp.float32),
        grid=(np_ // _BT,),
        in_specs=[
            pl.BlockSpec((_BT, 28, 28), lambda i: (i, 0, 0)),
            pl.BlockSpec((96, 1024), lambda i: (0, 0)),
            pl.BlockSpec((1, 1, 512), lambda i: (0, 0, 0)),
            pl.BlockSpec((1536, 1024), lambda i: (0, 0)),
            pl.BlockSpec((1, 1, 512), lambda i: (0, 0, 0)),
            pl.BlockSpec((4096, 128), lambda i: (0, 0)),
            pl.BlockSpec((1, 128), lambda i: (0, 0)),
            pl.BlockSpec((128, 10), lambda i: (0, 0)),
            pl.BlockSpec((1, 10), lambda i: (0, 0)),
        ],
        out_specs=pl.BlockSpec((_BT, 10), lambda i: (i, 0)),
        compiler_params=pltpu.CompilerParams(
            dimension_semantics=("arbitrary",),
            vmem_limit_bytes=56 * 1024 * 1024),
    )(xpad, b1, s1, b2, s2, fw1, fc1_b, fw2, fc2_b)
    return out[:n] if np_ != n else out
